# baseline (speedup 1.0000x reference)
import functools

import jax
import jax.numpy as jnp
from jax import lax
from jax.experimental import pallas as pl
from jax.experimental.pallas import tpu as pltpu
from jax.experimental.pallas import tpu_sc as plsc

N_NODES = 10000
H = 256
HH = 128
NUM_ET = 16
CHUNK = 112
N_CHUNKS = 90
EDGES_PER_SUB = N_CHUNKS * CHUNK
E_PAD = 16 * EDGES_PER_SUB
N_PAD = 10240

_ROW_BLK = 1000
_DEG_BLK = 2000


def _pre_body(x_ref, w1_ref, emb_ref, w2_ref, bm_ref, y_ref, t_ref):
    b = pl.program_id(0)
    y_ref[...] = jnp.dot(x_ref[...], w1_ref[...],
                         preferred_element_type=jnp.float32)

    @pl.when(b % 10 == 0)
    def _():
        t_ref[...] = jnp.dot(emb_ref[...], w2_ref[...],
                             preferred_element_type=jnp.float32) + bm_ref[...]


_pre = pl.pallas_call(
    _pre_body,
    grid=(20,),
    in_specs=[
        pl.BlockSpec((_ROW_BLK, H), lambda b: (b % 10, 0)),
        pl.BlockSpec((H, HH), lambda b: (0, b // 10)),
        pl.BlockSpec((NUM_ET, NUM_ET), lambda b: (0, 0)),
        pl.BlockSpec((NUM_ET, HH), lambda b: (0, b // 10)),
        pl.BlockSpec((1, HH), lambda b: (0, b // 10)),
    ],
    out_specs=[
        pl.BlockSpec((_ROW_BLK, HH), lambda b: (b, 0)),
        pl.BlockSpec((NUM_ET, HH), lambda b: (b // 10, 0)),
    ],
    out_shape=[
        jax.ShapeDtypeStruct((2 * N_NODES, HH), jnp.float32),
        jax.ShapeDtypeStruct((2 * NUM_ET, HH), jnp.float32),
    ],
)


def _deg_body(dst_ref, out_ref):
    i = pl.program_id(0)
    n = pl.num_programs(0)

    @pl.when(i == 0)
    def _():
        out_ref[...] = jnp.zeros_like(out_ref)

    d = dst_ref[...]
    hi = d // HH
    lo = d - hi * HH
    a = (hi == lax.broadcasted_iota(jnp.int32, (_DEG_BLK, N_PAD // HH), 1)
         ).astype(jnp.float32)
    b = (lo == lax.broadcasted_iota(jnp.int32, (_DEG_BLK, HH), 1)
         ).astype(jnp.float32)
    out_ref[...] += lax.dot_general(a, b, (((0,), (0,)), ((), ())),
                                    preferred_element_type=jnp.float32)

    @pl.when(i == n - 1)
    def _():
        out_ref[...] = 1.0 / jnp.maximum(out_ref[...], 1.0)


_deg = pl.pallas_call(
    _deg_body,
    grid=(160000 // _DEG_BLK,),
    in_specs=[pl.BlockSpec((_DEG_BLK, 1), lambda i: (i, 0))],
    out_specs=pl.BlockSpec((N_PAD // HH, HH), lambda i: (0, 0)),
    out_shape=jax.ShapeDtypeStruct((N_PAD // HH, HH), jnp.float32),
)


_SC_MESH = plsc.VectorSubcoreMesh(core_axis_name="c", subcore_axis_name="s")


@functools.partial(
    pl.kernel,
    mesh=_SC_MESH,
    out_type=jax.ShapeDtypeStruct((2 * N_NODES, HH), jnp.float32),
    scratch_types=[
        pltpu.VMEM((CHUNK, HH), jnp.float32),
        pltpu.VMEM((CHUNK, HH), jnp.float32),
        pltpu.VMEM((CHUNK, HH), jnp.float32),
        pltpu.VMEM((NUM_ET, HH), jnp.float32),
        pltpu.VMEM((CHUNK,), jnp.int32),
        pltpu.VMEM((CHUNK,), jnp.int32),
        pltpu.VMEM((CHUNK,), jnp.int32),
        pltpu.VMEM((CHUNK,), jnp.int32),
        pltpu.VMEM((CHUNK + 16,), jnp.int32),
        pltpu.VMEM((CHUNK + 16,), jnp.int32),
        pltpu.VMEM_SHARED((N_PAD, HH), jnp.float32),
        pltpu.SemaphoreType.DMA,
        pltpu.SemaphoreType.DMA,
        pltpu.SemaphoreType.DMA,
        pltpu.SemaphoreType.DMA,
    ],
)
def _sc_edges(y_hbm, t_hbm, src_hbm, dst_hbm, et_hbm, out_hbm,
              y0, y1, m, tloc, sb0, sb1, db0, db1, eb0, eb1, aggs,
              gs0, gs1, is0, is1):
    c = lax.axis_index("c")
    s = lax.axis_index("s")
    ybuf = (y0, y1)
    sbuf = (sb0, sb1)
    dbuf = (db0, db1)
    ebuf = (eb0, eb1)
    gsem = (gs0, gs1)
    isem = (is0, is1)

    zero16 = jnp.zeros((16,), jnp.float32)

    def zrow(r, carry):
        for j in range(HH // 16):
            m[r, pl.ds(j * 16, 16)] = zero16
        return carry

    lax.fori_loop(0, CHUNK, zrow, 0)

    rows_per_sub = N_PAD // 16
    for i in range(5):
        pltpu.sync_copy(m, aggs.at[pl.ds(s * rows_per_sub + i * CHUNK, CHUNK)])
    pltpu.sync_copy(m.at[pl.ds(0, rows_per_sub - 5 * CHUNK)],
                    aggs.at[pl.ds(s * rows_per_sub + 5 * CHUNK,
                                  rows_per_sub - 5 * CHUNK)])

    pltpu.sync_copy(t_hbm.at[pl.ds(c * NUM_ET, NUM_ET)], tloc)
    plsc.subcore_barrier()

    def src_base(g):
        return ((c * 16 + s) * N_CHUNKS + g) * CHUNK

    def ds_base(g):
        return (s * N_CHUNKS + g) * CHUNK

    def ifetch(g, b):
        pltpu.async_copy(src_hbm.at[pl.ds(src_base(g), CHUNK)], sbuf[b],
                         isem[b])
        pltpu.async_copy(dst_hbm.at[pl.ds(ds_base(g), CHUNK)], dbuf[b],
                         isem[b])
        pltpu.async_copy(et_hbm.at[pl.ds(ds_base(g), CHUNK)],
                         ebuf[b].at[pl.ds(0, CHUNK)], isem[b])

    def iwait(g, b):
        pltpu.make_async_copy(src_hbm.at[pl.ds(src_base(g), CHUNK)], sbuf[b],
                              isem[b]).wait()
        pltpu.make_async_copy(dst_hbm.at[pl.ds(ds_base(g), CHUNK)], dbuf[b],
                              isem[b]).wait()
        pltpu.make_async_copy(et_hbm.at[pl.ds(ds_base(g), CHUNK)],
                              ebuf[b].at[pl.ds(0, CHUNK)], isem[b]).wait()

    def gather(b):
        pltpu.async_copy(y_hbm.at[sbuf[b]], ybuf[b], gsem[b])

    def gwait(b):
        pltpu.make_async_copy(y_hbm.at[sbuf[b]], ybuf[b], gsem[b]).wait()

    def compute(b):
        yb = ybuf[b]
        eb = ebuf[b]

        @plsc.parallel_loop(0, CHUNK, unroll=8)
        def comp(r):
            e = eb[pl.ds(r, 16)][0]
            for j in range(HH // 16):
                sl = pl.ds(j * 16, 16)
                m[r, sl] = jnp.maximum(yb[r, sl] + tloc[e, sl],
                                       jnp.float32(0.0))

    ifetch(0, 0)
    ifetch(1, 1)
    iwait(0, 0)
    gather(0)

    def pair(i, cy):
        for b in range(2):
            g = 2 * i + b
            b1 = 1 - b
            iwait(lax.rem(g + 1, N_CHUNKS), b1)
            gather(b1)
            gwait(b)
            compute(b)
            pltpu.sync_copy(m, aggs.at[dbuf[b]], add=True)
            ifetch(lax.rem(g + 2, N_CHUNKS), b)
        return cy

    lax.fori_loop(0, N_CHUNKS // 2, pair, 0)

    gwait(0)
    iwait(1, 1)

    plsc.subcore_barrier()

    out_rows = 624
    pltpu.sync_copy(aggs.at[pl.ds(s * out_rows, out_rows)],
                    out_hbm.at[pl.ds(c * N_NODES + s * out_rows, out_rows)])

    @pl.when(s == 15)
    def _():
        tail = 16 * out_rows
        pltpu.sync_copy(aggs.at[pl.ds(tail, N_NODES - tail)],
                        out_hbm.at[pl.ds(c * N_NODES + tail, N_NODES - tail)])


def _post_body(x_ref, a0_ref, a1_ref, r_ref, w_ref, b_ref, g_ref, be_ref,
               o_ref):
    inv = r_ref[...]
    out = jnp.dot(a0_ref[...] * inv, w_ref[:HH, :],
                  preferred_element_type=jnp.float32)
    out = out + jnp.dot(a1_ref[...] * inv, w_ref[HH:, :],
                        preferred_element_type=jnp.float32)
    h = x_ref[...] + out + b_ref[...]
    mu = jnp.mean(h, axis=1, keepdims=True)
    d = h - mu
    var = jnp.mean(d * d, axis=1, keepdims=True)
    o_ref[...] = d * lax.rsqrt(var + 1e-5) * g_ref[...] + be_ref[...]


_post = pl.pallas_call(
    _post_body,
    grid=(10,),
    in_specs=[
        pl.BlockSpec((_ROW_BLK, H), lambda i: (i, 0)),
        pl.BlockSpec((_ROW_BLK, HH), lambda i: (i, 0)),
        pl.BlockSpec((_ROW_BLK, HH), lambda i: (i, 0)),
        pl.BlockSpec((_ROW_BLK, 1), lambda i: (i, 0)),
        pl.BlockSpec((H, H), lambda i: (0, 0)),
        pl.BlockSpec((1, H), lambda i: (0, 0)),
        pl.BlockSpec((1, H), lambda i: (0, 0)),
        pl.BlockSpec((1, H), lambda i: (0, 0)),
    ],
    out_specs=pl.BlockSpec((_ROW_BLK, H), lambda i: (i, 0)),
    out_shape=jax.ShapeDtypeStruct((N_NODES, H), jnp.float32),
)


def kernel(x, edge_index, edge_type, edge_emb, W_msg, b_msg,
           W_upd, b_upd, ln_gamma, ln_beta):
    src = edge_index[0].astype(jnp.int32)
    dst = edge_index[1].astype(jnp.int32)
    et = edge_type.astype(jnp.int32)
    e = src.shape[0]
    pad = E_PAD - e
    pr = jnp.arange(pad, dtype=jnp.int32)
    src_f = jnp.concatenate([src, pr % N_NODES])
    src_p = jnp.concatenate([src_f, src_f + N_NODES])
    dst_p = jnp.concatenate([dst, N_NODES + pr % (N_PAD - N_NODES)])
    et_p = jnp.concatenate([et, pr % NUM_ET])

    y_comb, t_comb = _pre(x, W_msg[:H], edge_emb, W_msg[H:],
                          b_msg.reshape(1, H))
    inv_deg = _deg(dst.reshape(e, 1)).reshape(-1)[:N_NODES].reshape(N_NODES, 1)
    out_sc = _sc_edges(y_comb, t_comb, src_p, dst_p, et_p)
    return _post(x, out_sc[:N_NODES], out_sc[N_NODES:], inv_deg, W_upd,
                 b_upd.reshape(1, H), ln_gamma.reshape(1, H),
                 ln_beta.reshape(1, H))


# speedup vs baseline: 1.0949x; 1.0949x over previous
import functools

import jax
import jax.numpy as jnp
from jax import lax
from jax.experimental import pallas as pl
from jax.experimental.pallas import tpu as pltpu
from jax.experimental.pallas import tpu_sc as plsc

N_NODES = 10000
H = 256
HH = 128
NUM_ET = 16
CHUNK = 80
N_CHUNKS = 128
EDGES_PER_SUB = N_CHUNKS * CHUNK
E_PAD = 16 * EDGES_PER_SUB
N_PAD = 10240

_ROW_BLK = 1000
_DEG_BLK = 2000


def _pre_body(x_ref, w1_ref, emb_ref, w2_ref, bm_ref, y_ref, t_ref):
    b = pl.program_id(0)
    y_ref[...] = jnp.dot(x_ref[...], w1_ref[...],
                         preferred_element_type=jnp.float32)

    @pl.when(b % 10 == 0)
    def _():
        t_ref[...] = jnp.dot(emb_ref[...], w2_ref[...],
                             preferred_element_type=jnp.float32) + bm_ref[...]


_pre = pl.pallas_call(
    _pre_body,
    grid=(20,),
    in_specs=[
        pl.BlockSpec((_ROW_BLK, H), lambda b: (b % 10, 0)),
        pl.BlockSpec((H, HH), lambda b: (0, b // 10)),
        pl.BlockSpec((NUM_ET, NUM_ET), lambda b: (0, 0)),
        pl.BlockSpec((NUM_ET, HH), lambda b: (0, b // 10)),
        pl.BlockSpec((1, HH), lambda b: (0, b // 10)),
    ],
    out_specs=[
        pl.BlockSpec((_ROW_BLK, HH), lambda b: (b, 0)),
        pl.BlockSpec((NUM_ET, HH), lambda b: (b // 10, 0)),
    ],
    out_shape=[
        jax.ShapeDtypeStruct((2 * N_NODES, HH), jnp.float32),
        jax.ShapeDtypeStruct((2 * NUM_ET, HH), jnp.float32),
    ],
)


def _deg_body(dst_ref, out_ref):
    i = pl.program_id(0)
    n = pl.num_programs(0)

    @pl.when(i == 0)
    def _():
        out_ref[...] = jnp.zeros_like(out_ref)

    d = dst_ref[...]
    hi = d // HH
    lo = d - hi * HH
    a = (hi == lax.broadcasted_iota(jnp.int32, (_DEG_BLK, N_PAD // HH), 1)
         ).astype(jnp.float32)
    b = (lo == lax.broadcasted_iota(jnp.int32, (_DEG_BLK, HH), 1)
         ).astype(jnp.float32)
    out_ref[...] += lax.dot_general(a, b, (((0,), (0,)), ((), ())),
                                    preferred_element_type=jnp.float32)

    @pl.when(i == n - 1)
    def _():
        out_ref[...] = 1.0 / jnp.maximum(out_ref[...], 1.0)


_deg = pl.pallas_call(
    _deg_body,
    grid=(160000 // _DEG_BLK,),
    in_specs=[pl.BlockSpec((_DEG_BLK, 1), lambda i: (i, 0))],
    out_specs=pl.BlockSpec((N_PAD // HH, HH), lambda i: (0, 0)),
    out_shape=jax.ShapeDtypeStruct((N_PAD // HH, HH), jnp.float32),
)


_SC_MESH = plsc.VectorSubcoreMesh(core_axis_name="c", subcore_axis_name="s")


@functools.partial(
    pl.kernel,
    mesh=_SC_MESH,
    out_type=jax.ShapeDtypeStruct((2 * N_NODES, HH), jnp.float32),
    scratch_types=[
        pltpu.VMEM((CHUNK, HH), jnp.float32),
        pltpu.VMEM((CHUNK, HH), jnp.float32),
        pltpu.VMEM((CHUNK, HH), jnp.float32),
        pltpu.VMEM((CHUNK, HH), jnp.float32),
        pltpu.VMEM((NUM_ET, HH), jnp.float32),
        pltpu.VMEM((CHUNK,), jnp.int32),
        pltpu.VMEM((CHUNK,), jnp.int32),
        pltpu.VMEM((CHUNK,), jnp.int32),
        pltpu.VMEM((CHUNK,), jnp.int32),
        pltpu.VMEM((CHUNK,), jnp.int32),
        pltpu.VMEM((CHUNK,), jnp.int32),
        pltpu.VMEM((CHUNK + 16,), jnp.int32),
        pltpu.VMEM((CHUNK + 16,), jnp.int32),
        pltpu.VMEM_SHARED((N_PAD, HH), jnp.float32),
        pltpu.SemaphoreType.DMA,
        pltpu.SemaphoreType.DMA,
        pltpu.SemaphoreType.DMA,
        pltpu.SemaphoreType.DMA,
        pltpu.SemaphoreType.DMA,
        pltpu.SemaphoreType.DMA,
    ],
)
def _sc_edges(y_hbm, t_hbm, src_hbm, dst_hbm, et_hbm, out_hbm,
              y0, y1, m0, m1, tloc, sb0, sb1, db0, db1, db2, db3,
              eb0, eb1, aggs, gs0, gs1, is0, is1, ss0, ss1):
    c = lax.axis_index("c")
    s = lax.axis_index("s")
    ybuf = (y0, y1)
    mbuf = (m0, m1)
    sbuf = (sb0, sb1)
    dbuf = (db0, db1, db2, db3)
    ebuf = (eb0, eb1)
    gsem = (gs0, gs1)
    isem = (is0, is1)
    ssem = (ss0, ss1)

    zero16 = jnp.zeros((16,), jnp.float32)

    def zrow(r, carry):
        for j in range(HH // 16):
            m0[r, pl.ds(j * 16, 16)] = zero16
            m1[r, pl.ds(j * 16, 16)] = zero16
        return carry

    lax.fori_loop(0, CHUNK, zrow, 0)

    rows_per_sub = N_PAD // 16
    for i in range(rows_per_sub // CHUNK):
        pltpu.sync_copy(m0, aggs.at[pl.ds(s * rows_per_sub + i * CHUNK,
                                          CHUNK)])

    pltpu.sync_copy(t_hbm.at[pl.ds(c * NUM_ET, NUM_ET)], tloc)
    plsc.subcore_barrier()

    def src_base(g):
        return ((c * 16 + s) * N_CHUNKS + g) * CHUNK

    def ds_base(g):
        return (s * N_CHUNKS + g) * CHUNK

    def ifetch(g, b, q):
        pltpu.async_copy(src_hbm.at[pl.ds(src_base(g), CHUNK)], sbuf[b],
                         isem[b])
        pltpu.async_copy(dst_hbm.at[pl.ds(ds_base(g), CHUNK)], dbuf[q],
                         isem[b])
        pltpu.async_copy(et_hbm.at[pl.ds(ds_base(g), CHUNK)],
                         ebuf[b].at[pl.ds(0, CHUNK)], isem[b])

    def iwait(g, b, q):
        pltpu.make_async_copy(src_hbm.at[pl.ds(src_base(g), CHUNK)], sbuf[b],
                              isem[b]).wait()
        pltpu.make_async_copy(dst_hbm.at[pl.ds(ds_base(g), CHUNK)], dbuf[q],
                              isem[b]).wait()
        pltpu.make_async_copy(et_hbm.at[pl.ds(ds_base(g), CHUNK)],
                              ebuf[b].at[pl.ds(0, CHUNK)], isem[b]).wait()

    def gather(b):
        pltpu.async_copy(y_hbm.at[sbuf[b]], ybuf[b], gsem[b])

    def gwait(b):
        pltpu.make_async_copy(y_hbm.at[sbuf[b]], ybuf[b], gsem[b]).wait()

    def scatter(b, q):
        pltpu.async_copy(mbuf[b], aggs.at[dbuf[q]], ssem[b], add=True)

    def swait(b, q):
        pltpu.make_async_copy(mbuf[b], aggs.at[dbuf[q]], ssem[b]).wait()

    def compute(b):
        yb = ybuf[b]
        eb = ebuf[b]
        mb = mbuf[b]

        @plsc.parallel_loop(0, CHUNK, unroll=4)
        def comp(r):
            e = eb[pl.ds(r, 16)][0]
            for j in range(HH // 16):
                sl = pl.ds(j * 16, 16)
                mb[r, sl] = jnp.maximum(yb[r, sl] + tloc[e, sl],
                                        jnp.float32(0.0))

    ifetch(0, 0, 0)
    ifetch(1, 1, 1)
    iwait(0, 0, 0)
    gather(0)
    full16 = jnp.full((16,), N_NODES, jnp.int32)
    for q in (2, 3):
        for j in range(CHUNK // 16):
            dbuf[q][pl.ds(j * 16, 16)] = full16
    scatter(0, 2)
    scatter(1, 3)

    def quad(i, cy):
        for k in range(4):
            g = 4 * i + k
            b = k % 2
            b1 = 1 - b
            iwait(lax.rem(g + 1, N_CHUNKS), b1, (k + 1) % 4)
            gather(b1)
            gwait(b)
            swait(b, (k + 2) % 4)
            compute(b)
            scatter(b, k)
            ifetch(lax.rem(g + 2, N_CHUNKS), b, (k + 2) % 4)
        return cy

    lax.fori_loop(0, N_CHUNKS // 4, quad, 0)

    gwait(0)
    iwait(1, 1, 1)
    swait(0, 2)
    swait(1, 3)

    plsc.subcore_barrier()

    out_rows = 624
    pltpu.sync_copy(aggs.at[pl.ds(s * out_rows, out_rows)],
                    out_hbm.at[pl.ds(c * N_NODES + s * out_rows, out_rows)])

    @pl.when(s == 15)
    def _():
        tail = 16 * out_rows
        pltpu.sync_copy(aggs.at[pl.ds(tail, N_NODES - tail)],
                        out_hbm.at[pl.ds(c * N_NODES + tail, N_NODES - tail)])


def _post_body(x_ref, a0_ref, a1_ref, r_ref, w_ref, b_ref, g_ref, be_ref,
               o_ref):
    inv = r_ref[...]
    out = jnp.dot(a0_ref[...] * inv, w_ref[:HH, :],
                  preferred_element_type=jnp.float32)
    out = out + jnp.dot(a1_ref[...] * inv, w_ref[HH:, :],
                        preferred_element_type=jnp.float32)
    h = x_ref[...] + out + b_ref[...]
    mu = jnp.mean(h, axis=1, keepdims=True)
    d = h - mu
    var = jnp.mean(d * d, axis=1, keepdims=True)
    o_ref[...] = d * lax.rsqrt(var + 1e-5) * g_ref[...] + be_ref[...]


_post = pl.pallas_call(
    _post_body,
    grid=(10,),
    in_specs=[
        pl.BlockSpec((_ROW_BLK, H), lambda i: (i, 0)),
        pl.BlockSpec((_ROW_BLK, HH), lambda i: (i, 0)),
        pl.BlockSpec((_ROW_BLK, HH), lambda i: (i, 0)),
        pl.BlockSpec((_ROW_BLK, 1), lambda i: (i, 0)),
        pl.BlockSpec((H, H), lambda i: (0, 0)),
        pl.BlockSpec((1, H), lambda i: (0, 0)),
        pl.BlockSpec((1, H), lambda i: (0, 0)),
        pl.BlockSpec((1, H), lambda i: (0, 0)),
    ],
    out_specs=pl.BlockSpec((_ROW_BLK, H), lambda i: (i, 0)),
    out_shape=jax.ShapeDtypeStruct((N_NODES, H), jnp.float32),
)


def kernel(x, edge_index, edge_type, edge_emb, W_msg, b_msg,
           W_upd, b_upd, ln_gamma, ln_beta):
    src = edge_index[0].astype(jnp.int32)
    dst = edge_index[1].astype(jnp.int32)
    et = edge_type.astype(jnp.int32)
    e = src.shape[0]
    pad = E_PAD - e
    pr = jnp.arange(pad, dtype=jnp.int32)
    src_f = jnp.concatenate([src, pr % N_NODES])
    src_p = jnp.concatenate([src_f, src_f + N_NODES])
    dst_p = jnp.concatenate([dst, N_NODES + pr % (N_PAD - N_NODES)])
    et_p = jnp.concatenate([et, pr % NUM_ET])

    y_comb, t_comb = _pre(x, W_msg[:H], edge_emb, W_msg[H:],
                          b_msg.reshape(1, H))
    inv_deg = _deg(dst.reshape(e, 1)).reshape(-1)[:N_NODES].reshape(N_NODES, 1)
    out_sc = _sc_edges(y_comb, t_comb, src_p, dst_p, et_p)
    return _post(x, out_sc[:N_NODES], out_sc[N_NODES:], inv_deg, W_upd,
                 b_upd.reshape(1, H), ln_gamma.reshape(1, H),
                 ln_beta.reshape(1, H))
